# Initial kernel scaffold; baseline (speedup 1.0000x reference)
import jax
import jax.numpy as jnp
from jax.experimental import pallas as pl


def kernel(story, C0, C1, C2, C3):
    raise NotImplementedError("write your pallas kernel here")


# speedup vs baseline: 9.1369x; 9.1369x over previous
import functools

import jax
import jax.numpy as jnp
from jax import lax
from jax.experimental import pallas as pl
from jax.experimental.pallas import tpu as pltpu
from jax.experimental.pallas import tpu_sc as plsc

VOCAB = 100000
D = 64
HOPS = 3
L_MEM = 50
B = 1024
T = 20
NTAB = HOPS + 1

LANES = 16
NC, NS = 2, 16
NW = NC * NS
B_PER_W = B // NW

CHUNK = 104
REAL = 100
NCHUNK = 10
LPAD = 64
NG = D // LANES


def _body(story_ref, c0_ref, c1_ref, c2_ref, c3_ref, o_ref, u_ref,
          idx_v, stage_v, e_v, p_v, uvec_v, o_v, uout_v, sem):
    tables = (c0_ref, c1_ref, c2_ref, c3_ref)
    wid = lax.axis_index("s") * NC + lax.axis_index("c")
    iota = lax.iota(jnp.int32, LANES)
    zeros = jnp.zeros((LANES,), jnp.float32)

    def batch_body(i, _):
        b = wid * B_PER_W + i
        pltpu.sync_copy(story_ref.at[b], idx_v)

        for h in range(NTAB):
            handles = [
                pltpu.async_copy(tables[h].at[idx_v.at[j]], stage_v.at[j], sem)
                for j in range(NCHUNK)
            ]
            for hd in handles:
                hd.wait()

            def seg_body(s, _):
                j = s // 5
                base = (s % 5) * T
                accs = [zeros] * NG
                for t in range(T):
                    for g in range(NG):
                        accs[g] = accs[g] + stage_v[j, base + t, pl.ds(g * LANES, LANES)]
                for g in range(NG):
                    e_v[h, s, pl.ds(g * LANES, LANES)] = accs[g]
                return _

            lax.fori_loop(0, L_MEM, seg_body, None)

        for g in range(NG):
            uvec_v[pl.ds(g * LANES, LANES)] = zeros
            uout_v[0, pl.ds(g * LANES, LANES)] = zeros

        for hop in range(HOPS):
            def score_body(d, carry):
                didx = jnp.full((LANES,), d, jnp.int32)
                ub = plsc.load_gather(uvec_v, [didx])
                hidx = jnp.full((LANES,), hop, jnp.int32)
                out = []
                for g in range(NG):
                    col = plsc.load_gather(e_v, [hidx, g * LANES + iota, didx])
                    out.append(carry[g] + col * ub)
                return tuple(out)

            scores = lax.fori_loop(0, D, score_body, (zeros,) * NG)

            valid = [g * LANES + iota < L_MEM for g in range(NG)]
            sm = [jnp.where(valid[g], scores[g], -1e30) for g in range(NG)]
            m = jnp.max(jnp.maximum(jnp.maximum(sm[0], sm[1]),
                                    jnp.maximum(sm[2], sm[3])))
            mb = jnp.full((LANES,), m, jnp.float32)
            es = [jnp.where(valid[g], jnp.exp(sm[g] - mb), 0.0) for g in range(NG)]
            tot = jnp.sum(es[0] + es[1] + es[2] + es[3])
            totv = jnp.full((LANES,), tot, jnp.float32)
            for g in range(NG):
                p_v[pl.ds(g * LANES, LANES)] = es[g] / totv

            def ok_body(l, carry):
                pb = plsc.load_gather(p_v, [jnp.full((LANES,), l, jnp.int32)])
                out = []
                for g in range(NG):
                    row = e_v[hop + 1, l, pl.ds(g * LANES, LANES)]
                    t = pb * row
                    if hop == HOPS - 1:
                        o_v[l, pl.ds(g * LANES, LANES)] = t
                    out.append(carry[g] + t)
                return tuple(out)

            ok = lax.fori_loop(0, L_MEM, ok_body, (zeros,) * NG)

            for g in range(NG):
                sl = pl.ds(g * LANES, LANES)
                unew = uvec_v[sl] + ok[g]
                uvec_v[sl] = unew
                uout_v[hop + 1, sl] = unew

        pltpu.sync_copy(o_v, o_ref.at[b])
        pltpu.sync_copy(uout_v, u_ref.at[b])
        return _

    lax.fori_loop(0, B_PER_W, batch_body, None)


@jax.jit
def kernel(story, C0, C1, C2, C3):
    s = jnp.transpose(story, (1, 0, 2)).reshape(B, NCHUNK, REAL)
    s = jnp.pad(s, ((0, 0), (0, 0), (0, CHUNK - REAL)))

    mesh = plsc.VectorSubcoreMesh(
        core_axis_name="c", subcore_axis_name="s",
        num_cores=NC, num_subcores=NS,
    )
    o, u_bt = pl.kernel(
        _body,
        out_type=(
            jax.ShapeDtypeStruct((B, L_MEM, D), jnp.float32),
            jax.ShapeDtypeStruct((B, NTAB, D), jnp.float32),
        ),
        mesh=mesh,
        scratch_types=[
            pltpu.VMEM((NCHUNK, CHUNK), jnp.int32),
            pltpu.VMEM((NCHUNK, CHUNK, D), jnp.float32),
            pltpu.VMEM((NTAB, LPAD, D), jnp.float32),
            pltpu.VMEM((LPAD,), jnp.float32),
            pltpu.VMEM((D,), jnp.float32),
            pltpu.VMEM((L_MEM, D), jnp.float32),
            pltpu.VMEM((NTAB, D), jnp.float32),
            pltpu.SemaphoreType.DMA,
        ],
        compiler_params=pltpu.CompilerParams(
            needs_layout_passes=False, use_tc_tiling_on_sc=False),
        name="mem_nn_encoder_sc",
    )(s, C0, C1, C2, C3)
    return (o, jnp.transpose(u_bt, (1, 0, 2)))


# speedup vs baseline: 13.4005x; 1.4666x over previous
import functools

import jax
import jax.numpy as jnp
from jax import lax
from jax.experimental import pallas as pl
from jax.experimental.pallas import tpu as pltpu
from jax.experimental.pallas import tpu_sc as plsc

VOCAB = 100000
D = 64
HOPS = 3
L_MEM = 50
B = 1024
T = 20
NTAB = HOPS + 1

LANES = 16
NC, NS = 2, 16
NW = NC * NS
B_PER_W = B // NW

NIDX = L_MEM * T
NPAD = 1040
ROWS_A = 520
ROWS_B = 480
SEGS_A = ROWS_A // T
SEGS_B = ROWS_B // T
LPAD = 64
NG = D // LANES


def _body(story_ref, c0_ref, c1_ref, c2_ref, c3_ref, o_ref, u_ref,
          idx_v, stage_a, stage_b, e_v, p_v, uvec_v, o_v, uout_v,
          sem_a, sem_b):
    tables = (c0_ref, c1_ref, c2_ref, c3_ref)
    wid = lax.axis_index("s") * NC + lax.axis_index("c")
    iota = lax.iota(jnp.int32, LANES)
    zeros = jnp.zeros((LANES,), jnp.float32)

    def accumulate(stage, h, seg0, nseg):
        def seg_body(s, _):
            base = s * T
            acc0 = [zeros] * NG
            acc1 = [zeros] * NG
            for t in range(0, T, 2):
                for g in range(NG):
                    acc0[g] = acc0[g] + stage[base + t, pl.ds(g * LANES, LANES)]
                    acc1[g] = acc1[g] + stage[base + t + 1, pl.ds(g * LANES, LANES)]
            for g in range(NG):
                e_v[h, seg0 + s, pl.ds(g * LANES, LANES)] = acc0[g] + acc1[g]
            return _
        lax.fori_loop(0, nseg, seg_body, None)

    def batch_body(i, _):
        b = wid * B_PER_W + i
        pltpu.sync_copy(story_ref.at[b], idx_v)
        idx_a = idx_v.at[pl.ds(0, ROWS_A)]
        idx_b = idx_v.at[pl.ds(ROWS_A, ROWS_B)]

        h_a = pltpu.async_copy(tables[0].at[idx_a], stage_a, sem_a)
        for h in range(NTAB):
            h_a.wait()
            h_b = pltpu.async_copy(tables[h].at[idx_b], stage_b, sem_b)
            accumulate(stage_a, h, 0, SEGS_A)
            h_b.wait()
            if h < NTAB - 1:
                h_a = pltpu.async_copy(tables[h + 1].at[idx_a], stage_a, sem_a)
            accumulate(stage_b, h, SEGS_A, SEGS_B)

        for g in range(NG):
            uvec_v[pl.ds(g * LANES, LANES)] = zeros
            uout_v[0, pl.ds(g * LANES, LANES)] = zeros

        for hop in range(HOPS):
            def score_body(d, carry):
                didx = jnp.full((LANES,), d, jnp.int32)
                ub = plsc.load_gather(uvec_v, [didx])
                hidx = jnp.full((LANES,), hop, jnp.int32)
                out = []
                for g in range(NG):
                    col = plsc.load_gather(e_v, [hidx, g * LANES + iota, didx])
                    out.append(carry[g] + col * ub)
                return tuple(out)

            scores = lax.fori_loop(0, D, score_body, (zeros,) * NG)

            valid = [g * LANES + iota < L_MEM for g in range(NG)]
            sm = [jnp.where(valid[g], scores[g], -1e30) for g in range(NG)]
            m = jnp.max(jnp.maximum(jnp.maximum(sm[0], sm[1]),
                                    jnp.maximum(sm[2], sm[3])))
            mb = jnp.full((LANES,), m, jnp.float32)
            es = [jnp.where(valid[g], jnp.exp(sm[g] - mb), 0.0) for g in range(NG)]
            tot = jnp.sum(es[0] + es[1] + es[2] + es[3])
            totv = jnp.full((LANES,), tot, jnp.float32)
            for g in range(NG):
                p_v[pl.ds(g * LANES, LANES)] = es[g] / totv

            def ok_body(l, carry):
                pb = plsc.load_gather(p_v, [jnp.full((LANES,), l, jnp.int32)])
                out = []
                for g in range(NG):
                    row = e_v[hop + 1, l, pl.ds(g * LANES, LANES)]
                    t = pb * row
                    if hop == HOPS - 1:
                        o_v[l, pl.ds(g * LANES, LANES)] = t
                    out.append(carry[g] + t)
                return tuple(out)

            ok = lax.fori_loop(0, L_MEM, ok_body, (zeros,) * NG)

            for g in range(NG):
                sl = pl.ds(g * LANES, LANES)
                unew = uvec_v[sl] + ok[g]
                uvec_v[sl] = unew
                uout_v[hop + 1, sl] = unew

        pltpu.sync_copy(o_v, o_ref.at[b])
        pltpu.sync_copy(uout_v, u_ref.at[b])
        return _

    lax.fori_loop(0, B_PER_W, batch_body, None)


@jax.jit
def kernel(story, C0, C1, C2, C3):
    s = jnp.transpose(story, (1, 0, 2)).reshape(B, NIDX)
    s = jnp.pad(s, ((0, 0), (0, NPAD - NIDX)))

    mesh = plsc.VectorSubcoreMesh(
        core_axis_name="c", subcore_axis_name="s",
        num_cores=NC, num_subcores=NS,
    )
    o, u_bt = pl.kernel(
        _body,
        out_type=(
            jax.ShapeDtypeStruct((B, L_MEM, D), jnp.float32),
            jax.ShapeDtypeStruct((B, NTAB, D), jnp.float32),
        ),
        mesh=mesh,
        scratch_types=[
            pltpu.VMEM((NPAD,), jnp.int32),
            pltpu.VMEM((ROWS_A, D), jnp.float32),
            pltpu.VMEM((ROWS_B, D), jnp.float32),
            pltpu.VMEM((NTAB, LPAD, D), jnp.float32),
            pltpu.VMEM((LPAD,), jnp.float32),
            pltpu.VMEM((D,), jnp.float32),
            pltpu.VMEM((L_MEM, D), jnp.float32),
            pltpu.VMEM((NTAB, D), jnp.float32),
            pltpu.SemaphoreType.DMA,
            pltpu.SemaphoreType.DMA,
        ],
        compiler_params=pltpu.CompilerParams(
            needs_layout_passes=False, use_tc_tiling_on_sc=False),
        name="mem_nn_encoder_sc",
    )(s, C0, C1, C2, C3)
    return (o, jnp.transpose(u_bt, (1, 0, 2)))


# speedup vs baseline: 13.9533x; 1.0413x over previous
import functools

import jax
import jax.numpy as jnp
from jax import lax
from jax.experimental import pallas as pl
from jax.experimental.pallas import tpu as pltpu
from jax.experimental.pallas import tpu_sc as plsc

VOCAB = 100000
D = 64
HOPS = 3
L_MEM = 50
B = 1024
T = 20
NTAB = HOPS + 1

LANES = 16
NC, NS = 2, 16
NW = NC * NS
B_PER_W = B // NW

NIDX = L_MEM * T
NPAD = 1040
ROWS_A = 520
ROWS_B = 480
SEGS_A = ROWS_A // T
SEGS_B = ROWS_B // T
NFLAT = (NIDX + LANES - 1) // LANES
LPAD = 64
NG = D // LANES


def _body(story_ref, c0_ref, c1_ref, c2_ref, c3_ref, o_ref, u_ref,
          blk_v, idx_v, stage_a, stage_b, e_v, p_v, uvec_v, o_v, uout_v,
          sem_a, sem_b):
    tables = (c0_ref, c1_ref, c2_ref, c3_ref)
    wid = lax.axis_index("s") * NC + lax.axis_index("c")
    iota = lax.iota(jnp.int32, LANES)
    zeros = jnp.zeros((LANES,), jnp.float32)

    pltpu.sync_copy(story_ref.at[:, pl.ds(wid * B_PER_W, B_PER_W), :], blk_v)

    def flatten(i, slot):
        def fb(j, _):
            k = jnp.minimum(j * LANES + iota, NIDX - 1)
            l = k // T
            t = k - l * T
            v = plsc.load_gather(blk_v, [l, jnp.full((LANES,), i, jnp.int32), t])
            idx_v[slot, pl.ds(j * LANES, LANES)] = v
            return _
        lax.fori_loop(0, NFLAT, fb, None)

    def accumulate(stage, h, seg0, nseg):
        def seg_body(s, _):
            base = s * T
            acc0 = [zeros] * NG
            acc1 = [zeros] * NG
            for t in range(0, T, 2):
                for g in range(NG):
                    acc0[g] = acc0[g] + stage[base + t, pl.ds(g * LANES, LANES)]
                    acc1[g] = acc1[g] + stage[base + t + 1, pl.ds(g * LANES, LANES)]
            for g in range(NG):
                e_v[h, seg0 + s, pl.ds(g * LANES, LANES)] = acc0[g] + acc1[g]
            return _
        lax.fori_loop(0, nseg, seg_body, None)

    def drain_a(h, par):
        pltpu.make_async_copy(
            tables[h].at[idx_v.at[par, pl.ds(0, ROWS_A)]], stage_a, sem_a
        ).wait()

    flatten(jnp.int32(0), 0)
    pltpu.async_copy(tables[0].at[idx_v.at[0, pl.ds(0, ROWS_A)]], stage_a, sem_a)

    def batch_body(i, _):
        b = wid * B_PER_W + i
        par = lax.rem(i, 2)
        idx_a = idx_v.at[par, pl.ds(0, ROWS_A)]
        idx_b = idx_v.at[par, pl.ds(ROWS_A, ROWS_B)]

        for h in range(NTAB):
            drain_a(h, par)
            h_b = pltpu.async_copy(tables[h].at[idx_b], stage_b, sem_b)
            accumulate(stage_a, h, 0, SEGS_A)
            h_b.wait()
            if h < NTAB - 1:
                pltpu.async_copy(tables[h + 1].at[idx_a], stage_a, sem_a)
            accumulate(stage_b, h, SEGS_A, SEGS_B)

        inext = jnp.minimum(i + 1, B_PER_W - 1)
        flatten(inext, 1 - par)
        pltpu.async_copy(
            tables[0].at[idx_v.at[1 - par, pl.ds(0, ROWS_A)]], stage_a, sem_a)

        for g in range(NG):
            uvec_v[pl.ds(g * LANES, LANES)] = zeros
            uout_v[0, pl.ds(g * LANES, LANES)] = zeros

        for hop in range(HOPS):
            def score_body(j, carry):
                out = list(carry)
                for u_ in range(2):
                    d = 2 * j + u_
                    didx = jnp.full((LANES,), d, jnp.int32)
                    ub = plsc.load_gather(uvec_v, [didx])
                    hidx = jnp.full((LANES,), hop, jnp.int32)
                    for g in range(NG):
                        col = plsc.load_gather(e_v, [hidx, g * LANES + iota, didx])
                        out[g] = out[g] + col * ub
                return tuple(out)

            scores = lax.fori_loop(0, D // 2, score_body, (zeros,) * NG)

            valid = [g * LANES + iota < L_MEM for g in range(NG)]
            sm = [jnp.where(valid[g], scores[g], -1e30) for g in range(NG)]
            m = jnp.max(jnp.maximum(jnp.maximum(sm[0], sm[1]),
                                    jnp.maximum(sm[2], sm[3])))
            mb = jnp.full((LANES,), m, jnp.float32)
            es = [jnp.where(valid[g], jnp.exp(sm[g] - mb), 0.0) for g in range(NG)]
            tot = jnp.sum(es[0] + es[1] + es[2] + es[3])
            totv = jnp.full((LANES,), tot, jnp.float32)
            for g in range(NG):
                p_v[pl.ds(g * LANES, LANES)] = es[g] / totv

            def ok_body(j, carry):
                out = list(carry)
                for u_ in range(2):
                    l = 2 * j + u_
                    pb = plsc.load_gather(p_v, [jnp.full((LANES,), l, jnp.int32)])
                    for g in range(NG):
                        row = e_v[hop + 1, l, pl.ds(g * LANES, LANES)]
                        t = pb * row
                        if hop == HOPS - 1:
                            o_v[l, pl.ds(g * LANES, LANES)] = t
                        out[g] = out[g] + t
                return tuple(out)

            ok = lax.fori_loop(0, L_MEM // 2, ok_body, (zeros,) * NG)

            for g in range(NG):
                sl = pl.ds(g * LANES, LANES)
                unew = uvec_v[sl] + ok[g]
                uvec_v[sl] = unew
                uout_v[hop + 1, sl] = unew

        pltpu.sync_copy(o_v, o_ref.at[b])
        pltpu.sync_copy(uout_v, u_ref.at[b])
        return _

    lax.fori_loop(0, B_PER_W, batch_body, None)
    drain_a(0, jnp.int32(0))


@jax.jit
def kernel(story, C0, C1, C2, C3):
    mesh = plsc.VectorSubcoreMesh(
        core_axis_name="c", subcore_axis_name="s",
        num_cores=NC, num_subcores=NS,
    )
    o, u_bt = pl.kernel(
        _body,
        out_type=(
            jax.ShapeDtypeStruct((B, L_MEM, D), jnp.float32),
            jax.ShapeDtypeStruct((B, NTAB, D), jnp.float32),
        ),
        mesh=mesh,
        scratch_types=[
            pltpu.VMEM((L_MEM, B_PER_W, T), jnp.int32),
            pltpu.VMEM((2, NPAD), jnp.int32),
            pltpu.VMEM((ROWS_A, D), jnp.float32),
            pltpu.VMEM((ROWS_B, D), jnp.float32),
            pltpu.VMEM((NTAB, LPAD, D), jnp.float32),
            pltpu.VMEM((LPAD,), jnp.float32),
            pltpu.VMEM((D,), jnp.float32),
            pltpu.VMEM((L_MEM, D), jnp.float32),
            pltpu.VMEM((NTAB, D), jnp.float32),
            pltpu.SemaphoreType.DMA,
            pltpu.SemaphoreType.DMA,
        ],
        compiler_params=pltpu.CompilerParams(
            needs_layout_passes=False, use_tc_tiling_on_sc=False),
        name="mem_nn_encoder_sc",
    )(story, C0, C1, C2, C3)
    return (o, jnp.transpose(u_bt, (1, 0, 2)))


# speedup vs baseline: 18.1805x; 1.3030x over previous
import functools

import jax
import jax.numpy as jnp
from jax import lax
from jax.experimental import pallas as pl
from jax.experimental.pallas import tpu as pltpu
from jax.experimental.pallas import tpu_sc as plsc

VOCAB = 100000
D = 64
HOPS = 3
L_MEM = 50
B = 1024
T = 20
NTAB = 3

LANES = 16
NC, NS = 2, 16
NW = NC * NS
B_PER_W = B // NW

NIDX = L_MEM * T
NPAD = 1040
ROWS_A = 520
ROWS_B = 480
SEGS_A = ROWS_A // T
SEGS_B = ROWS_B // T
NFLAT = (NIDX + LANES - 1) // LANES
LPAD = 64
NG = D // LANES


def _body(story_ref, c1_ref, c2_ref, c3_ref, o_ref, u_ref,
          blk_v, idx_v, stage_a, stage_b, e_v, p_v, uvec_v, o_v, uout_v,
          sem_a, sem_b):
    tables = (c1_ref, c2_ref, c3_ref)
    wid = lax.axis_index("s") * NC + lax.axis_index("c")
    iota = lax.iota(jnp.int32, LANES)
    zeros = jnp.zeros((LANES,), jnp.float32)

    pltpu.sync_copy(story_ref.at[:, pl.ds(wid * B_PER_W, B_PER_W), :], blk_v)

    def flatten(i, slot):
        def fb(j, _):
            k = jnp.minimum(j * LANES + iota, NIDX - 1)
            l = k // T
            t = k - l * T
            v = plsc.load_gather(blk_v, [l, jnp.full((LANES,), i, jnp.int32), t])
            idx_v[slot, pl.ds(j * LANES, LANES)] = v
            return _
        lax.fori_loop(0, NFLAT, fb, None)

    def accumulate(stage, h, seg0, nseg):
        def seg_body(s, _):
            base = s * T
            acc0 = [zeros] * NG
            acc1 = [zeros] * NG
            for t in range(0, T, 2):
                for g in range(NG):
                    acc0[g] = acc0[g] + stage[base + t, pl.ds(g * LANES, LANES)]
                    acc1[g] = acc1[g] + stage[base + t + 1, pl.ds(g * LANES, LANES)]
            for g in range(NG):
                e_v[h, seg0 + s, pl.ds(g * LANES, LANES)] = acc0[g] + acc1[g]
            return _
        lax.fori_loop(0, nseg, seg_body, None)

    def drain_a(h, par):
        pltpu.make_async_copy(
            tables[h].at[idx_v.at[par, pl.ds(0, ROWS_A)]], stage_a, sem_a
        ).wait()

    flatten(jnp.int32(0), 0)
    pltpu.async_copy(tables[0].at[idx_v.at[0, pl.ds(0, ROWS_A)]], stage_a, sem_a)

    def batch_body(i, _):
        b = wid * B_PER_W + i
        par = lax.rem(i, 2)
        idx_a = idx_v.at[par, pl.ds(0, ROWS_A)]
        idx_b = idx_v.at[par, pl.ds(ROWS_A, ROWS_B)]

        for h in range(NTAB):
            drain_a(h, par)
            h_b = pltpu.async_copy(tables[h].at[idx_b], stage_b, sem_b)
            accumulate(stage_a, h, 0, SEGS_A)
            h_b.wait()
            if h < NTAB - 1:
                pltpu.async_copy(tables[h + 1].at[idx_a], stage_a, sem_a)
            accumulate(stage_b, h, SEGS_A, SEGS_B)

        inext = jnp.minimum(i + 1, B_PER_W - 1)
        flatten(inext, 1 - par)
        pltpu.async_copy(
            tables[0].at[idx_v.at[1 - par, pl.ds(0, ROWS_A)]], stage_a, sem_a)

        def mean_body(j, carry):
            out = list(carry)
            for u_ in range(2):
                l = 2 * j + u_
                for g in range(NG):
                    out[g] = out[g] + e_v[0, l, pl.ds(g * LANES, LANES)]
            return tuple(out)

        ok = lax.fori_loop(0, L_MEM // 2, mean_body, (zeros,) * NG)
        inv_l = jnp.full((LANES,), 1.0 / L_MEM, jnp.float32)
        for g in range(NG):
            sl = pl.ds(g * LANES, LANES)
            uout_v[0, sl] = zeros
            u1 = ok[g] * inv_l
            uvec_v[sl] = u1
            uout_v[1, sl] = u1

        for hop in range(1, HOPS):
            def score_body(j, carry):
                out = list(carry)
                for u_ in range(2):
                    d = 2 * j + u_
                    didx = jnp.full((LANES,), d, jnp.int32)
                    ub = plsc.load_gather(uvec_v, [didx])
                    hidx = jnp.full((LANES,), hop - 1, jnp.int32)
                    for g in range(NG):
                        col = plsc.load_gather(e_v, [hidx, g * LANES + iota, didx])
                        out[g] = out[g] + col * ub
                return tuple(out)

            scores = lax.fori_loop(0, D // 2, score_body, (zeros,) * NG)

            valid = [g * LANES + iota < L_MEM for g in range(NG)]
            sm = [jnp.where(valid[g], scores[g], -1e30) for g in range(NG)]
            m = jnp.max(jnp.maximum(jnp.maximum(sm[0], sm[1]),
                                    jnp.maximum(sm[2], sm[3])))
            mb = jnp.full((LANES,), m, jnp.float32)
            es = [jnp.where(valid[g], jnp.exp(sm[g] - mb), 0.0) for g in range(NG)]
            tot = jnp.sum(es[0] + es[1] + es[2] + es[3])
            totv = jnp.full((LANES,), tot, jnp.float32)
            for g in range(NG):
                p_v[pl.ds(g * LANES, LANES)] = es[g] / totv

            def ok_body(j, carry):
                out = list(carry)
                for u_ in range(2):
                    l = 2 * j + u_
                    pb = plsc.load_gather(p_v, [jnp.full((LANES,), l, jnp.int32)])
                    for g in range(NG):
                        row = e_v[hop, l, pl.ds(g * LANES, LANES)]
                        t = pb * row
                        if hop == HOPS - 1:
                            o_v[l, pl.ds(g * LANES, LANES)] = t
                        out[g] = out[g] + t
                return tuple(out)

            ok = lax.fori_loop(0, L_MEM // 2, ok_body, (zeros,) * NG)

            for g in range(NG):
                sl = pl.ds(g * LANES, LANES)
                unew = uvec_v[sl] + ok[g]
                uvec_v[sl] = unew
                uout_v[hop + 1, sl] = unew

        pltpu.sync_copy(o_v, o_ref.at[b])
        pltpu.sync_copy(uout_v, u_ref.at[b])
        return _

    lax.fori_loop(0, B_PER_W, batch_body, None)
    drain_a(0, jnp.int32(0))


@jax.jit
def kernel(story, C0, C1, C2, C3):
    mesh = plsc.VectorSubcoreMesh(
        core_axis_name="c", subcore_axis_name="s",
        num_cores=NC, num_subcores=NS,
    )
    o, u_bt = pl.kernel(
        _body,
        out_type=(
            jax.ShapeDtypeStruct((B, L_MEM, D), jnp.float32),
            jax.ShapeDtypeStruct((B, HOPS + 1, D), jnp.float32),
        ),
        mesh=mesh,
        scratch_types=[
            pltpu.VMEM((L_MEM, B_PER_W, T), jnp.int32),
            pltpu.VMEM((2, NPAD), jnp.int32),
            pltpu.VMEM((ROWS_A, D), jnp.float32),
            pltpu.VMEM((ROWS_B, D), jnp.float32),
            pltpu.VMEM((NTAB, LPAD, D), jnp.float32),
            pltpu.VMEM((LPAD,), jnp.float32),
            pltpu.VMEM((D,), jnp.float32),
            pltpu.VMEM((L_MEM, D), jnp.float32),
            pltpu.VMEM((HOPS + 1, D), jnp.float32),
            pltpu.SemaphoreType.DMA,
            pltpu.SemaphoreType.DMA,
        ],
        compiler_params=pltpu.CompilerParams(
            needs_layout_passes=False, use_tc_tiling_on_sc=False),
        name="mem_nn_encoder_sc",
    )(story, C1, C2, C3)
    return (o, jnp.transpose(u_bt, (1, 0, 2)))


# speedup vs baseline: 18.2950x; 1.0063x over previous
import functools

import jax
import jax.numpy as jnp
from jax import lax
from jax.experimental import pallas as pl
from jax.experimental.pallas import tpu as pltpu
from jax.experimental.pallas import tpu_sc as plsc

VOCAB = 100000
D = 64
HOPS = 3
L_MEM = 50
B = 1024
T = 20
NTAB = 3

LANES = 16
NC, NS = 2, 16
NW = NC * NS
B_PER_W = B // NW

NIDX = L_MEM * T
NPAD = 1040
ROWS_A = 520
ROWS_B = 480
SEGS_A = ROWS_A // T
SEGS_B = ROWS_B // T
NFLAT = (NIDX + LANES - 1) // LANES
LPAD = 64
NG = D // LANES


def _body(story_ref, c1_ref, c2_ref, c3_ref, o_ref, u_ref,
          blk_v, idx_v, stage_a, stage_b, e_v, p_v, uvec_v, o_v, uout_v,
          sem_a, sem_b, sem_o, sem_u):
    tables = (c1_ref, c2_ref, c3_ref)
    wid = lax.axis_index("s") * NC + lax.axis_index("c")
    iota = lax.iota(jnp.int32, LANES)
    zeros = jnp.zeros((LANES,), jnp.float32)

    pltpu.sync_copy(story_ref.at[:, pl.ds(wid * B_PER_W, B_PER_W), :], blk_v)

    def flatten(i, slot):
        def fb(j, _):
            k = jnp.minimum(j * LANES + iota, NIDX - 1)
            l = k // T
            t = k - l * T
            v = plsc.load_gather(blk_v, [l, jnp.full((LANES,), i, jnp.int32), t])
            idx_v[slot, pl.ds(j * LANES, LANES)] = v
            return _
        lax.fori_loop(0, NFLAT, fb, None)

    def accumulate(stage, h, seg0, nseg):
        def seg_body(sj, _):
            for u_ in range(2):
                s = 2 * sj + u_
                base = s * T
                acc0 = [zeros] * NG
                acc1 = [zeros] * NG
                for t in range(0, T, 2):
                    for g in range(NG):
                        acc0[g] = acc0[g] + stage[base + t, pl.ds(g * LANES, LANES)]
                        acc1[g] = acc1[g] + stage[base + t + 1, pl.ds(g * LANES, LANES)]
                for g in range(NG):
                    e_v[h, seg0 + s, pl.ds(g * LANES, LANES)] = acc0[g] + acc1[g]
            return _
        lax.fori_loop(0, nseg // 2, seg_body, None)

    def drain_a(h, par):
        pltpu.make_async_copy(
            tables[h].at[idx_v.at[par, pl.ds(0, ROWS_A)]], stage_a, sem_a
        ).wait()

    flatten(jnp.int32(0), 0)
    pltpu.async_copy(tables[0].at[idx_v.at[0, pl.ds(0, ROWS_A)]], stage_a, sem_a)

    def batch_body(i, _):
        b = wid * B_PER_W + i
        par = lax.rem(i, 2)

        @pl.when(i >= 2)
        def _drain_outputs():
            pltpu.make_async_copy(o_v.at[par], o_ref.at[b], sem_o).wait()
            pltpu.make_async_copy(uout_v.at[par], u_ref.at[b], sem_u).wait()

        idx_a = idx_v.at[par, pl.ds(0, ROWS_A)]
        idx_b = idx_v.at[par, pl.ds(ROWS_A, ROWS_B)]

        for h in range(NTAB):
            drain_a(h, par)
            h_b = pltpu.async_copy(tables[h].at[idx_b], stage_b, sem_b)
            accumulate(stage_a, h, 0, SEGS_A)
            h_b.wait()
            if h < NTAB - 1:
                pltpu.async_copy(tables[h + 1].at[idx_a], stage_a, sem_a)
            accumulate(stage_b, h, SEGS_A, SEGS_B)

        inext = jnp.minimum(i + 1, B_PER_W - 1)
        flatten(inext, 1 - par)
        pltpu.async_copy(
            tables[0].at[idx_v.at[1 - par, pl.ds(0, ROWS_A)]], stage_a, sem_a)

        def mean_body(j, carry):
            out = list(carry)
            for u_ in range(2):
                l = 2 * j + u_
                for g in range(NG):
                    out[g] = out[g] + e_v[0, l, pl.ds(g * LANES, LANES)]
            return tuple(out)

        ok = lax.fori_loop(0, L_MEM // 2, mean_body, (zeros,) * NG)
        inv_l = jnp.full((LANES,), 1.0 / L_MEM, jnp.float32)
        for g in range(NG):
            sl = pl.ds(g * LANES, LANES)
            uout_v[par, 0, sl] = zeros
            u1 = ok[g] * inv_l
            uvec_v[sl] = u1
            uout_v[par, 1, sl] = u1

        for hop in range(1, HOPS):
            def score_body(j, carry):
                out = list(carry)
                for u_ in range(2):
                    d = 2 * j + u_
                    didx = jnp.full((LANES,), d, jnp.int32)
                    ub = plsc.load_gather(uvec_v, [didx])
                    hidx = jnp.full((LANES,), hop - 1, jnp.int32)
                    for g in range(NG):
                        col = plsc.load_gather(e_v, [hidx, g * LANES + iota, didx])
                        out[g] = out[g] + col * ub
                return tuple(out)

            scores = lax.fori_loop(0, D // 2, score_body, (zeros,) * NG)

            valid = [g * LANES + iota < L_MEM for g in range(NG)]
            sm = [jnp.where(valid[g], scores[g], -1e30) for g in range(NG)]
            m = jnp.max(jnp.maximum(jnp.maximum(sm[0], sm[1]),
                                    jnp.maximum(sm[2], sm[3])))
            mb = jnp.full((LANES,), m, jnp.float32)
            es = [jnp.where(valid[g], jnp.exp(sm[g] - mb), 0.0) for g in range(NG)]
            tot = jnp.sum(es[0] + es[1] + es[2] + es[3])
            totv = jnp.full((LANES,), tot, jnp.float32)
            for g in range(NG):
                p_v[pl.ds(g * LANES, LANES)] = es[g] / totv

            def ok_body(j, carry):
                out = list(carry)
                for u_ in range(2):
                    l = 2 * j + u_
                    pb = plsc.load_gather(p_v, [jnp.full((LANES,), l, jnp.int32)])
                    for g in range(NG):
                        row = e_v[hop, l, pl.ds(g * LANES, LANES)]
                        t = pb * row
                        if hop == HOPS - 1:
                            o_v[par, l, pl.ds(g * LANES, LANES)] = t
                        out[g] = out[g] + t
                return tuple(out)

            ok = lax.fori_loop(0, L_MEM // 2, ok_body, (zeros,) * NG)

            for g in range(NG):
                sl = pl.ds(g * LANES, LANES)
                unew = uvec_v[sl] + ok[g]
                uvec_v[sl] = unew
                uout_v[par, hop + 1, sl] = unew

        pltpu.async_copy(o_v.at[par], o_ref.at[b], sem_o)
        pltpu.async_copy(uout_v.at[par], u_ref.at[b], sem_u)
        return _

    lax.fori_loop(0, B_PER_W, batch_body, None)
    drain_a(0, jnp.int32(0))
    for _k in range(2):
        pltpu.make_async_copy(o_v.at[_k], o_ref.at[wid * B_PER_W + _k], sem_o).wait()
        pltpu.make_async_copy(uout_v.at[_k], u_ref.at[wid * B_PER_W + _k], sem_u).wait()


@jax.jit
def kernel(story, C0, C1, C2, C3):
    mesh = plsc.VectorSubcoreMesh(
        core_axis_name="c", subcore_axis_name="s",
        num_cores=NC, num_subcores=NS,
    )
    o, u_bt = pl.kernel(
        _body,
        out_type=(
            jax.ShapeDtypeStruct((B, L_MEM, D), jnp.float32),
            jax.ShapeDtypeStruct((B, HOPS + 1, D), jnp.float32),
        ),
        mesh=mesh,
        scratch_types=[
            pltpu.VMEM((L_MEM, B_PER_W, T), jnp.int32),
            pltpu.VMEM((2, NPAD), jnp.int32),
            pltpu.VMEM((ROWS_A, D), jnp.float32),
            pltpu.VMEM((ROWS_B, D), jnp.float32),
            pltpu.VMEM((NTAB, LPAD, D), jnp.float32),
            pltpu.VMEM((LPAD,), jnp.float32),
            pltpu.VMEM((D,), jnp.float32),
            pltpu.VMEM((2, L_MEM, D), jnp.float32),
            pltpu.VMEM((2, HOPS + 1, D), jnp.float32),
            pltpu.SemaphoreType.DMA,
            pltpu.SemaphoreType.DMA,
            pltpu.SemaphoreType.DMA,
            pltpu.SemaphoreType.DMA,
        ],
        compiler_params=pltpu.CompilerParams(
            needs_layout_passes=False, use_tc_tiling_on_sc=False),
        name="mem_nn_encoder_sc",
    )(story, C1, C2, C3)
    return (o, jnp.transpose(u_bt, (1, 0, 2)))
